# baseline (speedup 1.0000x reference)
import functools

import jax
import jax.numpy as jnp
from jax import lax
from jax.experimental import pallas as pl
from jax.experimental.pallas import tpu as pltpu
from jax.experimental.pallas import tpu_sc as plsc

_SX = 16.0 / 255.0
_INV_SX = 255.0 / 16.0

_HW = 1024
_C = 64
_B = 1024
_NW = 32
_HW_PER_W = _HW // _NW
_BCH = 256
_NBCH = _B // _BCH

_mesh = plsc.VectorSubcoreMesh(core_axis_name="c", subcore_axis_name="s")


@functools.partial(
    pl.kernel,
    mesh=_mesh,
    out_type=jax.ShapeDtypeStruct((_HW, _C, _B), jnp.float32),
    scratch_types=[
        pltpu.VMEM((_C, _BCH), jnp.float32),
        pltpu.VMEM((_C, _BCH), jnp.float32),
    ],
)
def _sc_kernel(x_hbm, o_hbm, ibuf, obuf):
    wid = lax.axis_index("s") * 2 + lax.axis_index("c")
    hw0 = wid * _HW_PER_W

    def do_chunk(hw, bc):
        pltpu.sync_copy(x_hbm.at[hw, :, pl.ds(bc * _BCH, _BCH)], ibuf)

        def one_group(j, _):
            ds = pl.ds(j * 16, 16)
            m = ibuf[0, ds]
            for c in range(1, _C):
                m = jnp.maximum(m, ibuf[c, ds])
            s = jnp.zeros((16,), jnp.float32)
            for c in range(_C):
                t = (ibuf[c, ds] - m) * _INV_SX
                q = (t - 0.5).astype(jnp.int32).astype(jnp.float32)
                e = jnp.exp(q * _SX)
                obuf[c, ds] = e
                s = s + e
            r = 255.0 / s
            for c in range(_C):
                y = obuf[c, ds] * r
                qy = (y + 0.5).astype(jnp.int32).astype(jnp.float32)
                obuf[c, ds] = qy * (1.0 / 255.0)
            return 0

        lax.fori_loop(0, _BCH // 16, one_group, 0)
        pltpu.sync_copy(obuf, o_hbm.at[hw, :, pl.ds(bc * _BCH, _BCH)])

    def row_body(i, _):
        def ch_body(bc, _2):
            do_chunk(hw0 + i, bc)
            return 0

        lax.fori_loop(0, _NBCH, ch_body, 0)
        return 0

    lax.fori_loop(0, _HW_PER_W, row_body, 0)


def kernel(inputs):
    b, h, w, w2 = inputs.shape
    xt = jnp.transpose(inputs, (1, 2, 3, 0)).reshape(h * w, w2, b)
    out = _sc_kernel(xt)
    return jnp.transpose(out.reshape(h, w, w2, b), (3, 0, 1, 2))


# speedup vs baseline: 23.2804x; 23.2804x over previous
import jax
import jax.numpy as jnp
from jax.experimental import pallas as pl
from jax.experimental.pallas import tpu as pltpu

_SX = 16.0 / 255.0
_INV_SX = 255.0 / 16.0
_C = _SX * 1.4426950408889634


def _body(x_ref, o_ref):
    x = x_ref[...]
    m = jnp.max(x, axis=1, keepdims=True)
    q = jnp.round((x - m) * _INV_SX)
    e = jnp.exp2(q * _C)
    s = jnp.sum(e, axis=1, keepdims=True)
    r = 255.0 / s
    o_ref[...] = jnp.round(e * r) * (1.0 / 255.0)


def kernel(inputs):
    b, h, w, w2 = inputs.shape
    xt = jnp.transpose(inputs, (1, 2, 3, 0)).reshape(h * w, w2, b)
    blk = 16
    out = pl.pallas_call(
        _body,
        grid=(xt.shape[0] // blk,),
        in_specs=[pl.BlockSpec((blk, w2, b), lambda i: (i, 0, 0))],
        out_specs=pl.BlockSpec((blk, w2, b), lambda i: (i, 0, 0)),
        out_shape=jax.ShapeDtypeStruct(xt.shape, xt.dtype),
    )(xt)
    return jnp.transpose(out.reshape(h, w, w2, b), (3, 0, 1, 2))
